# Initial kernel scaffold; baseline (speedup 1.0000x reference)
import jax
import jax.numpy as jnp
from jax.experimental import pallas as pl


def kernel(sentences, emb_table, fc_w, fc_b):
    raise NotImplementedError("write your pallas kernel here")


# speedup vs baseline: 1.0438x; 1.0438x over previous
import functools

import jax
import jax.numpy as jnp
from jax import lax
from jax.experimental import pallas as pl
from jax.experimental.pallas import tpu as pltpu
from jax.experimental.pallas import tpu_sc as plsc

_LANES = 16


@functools.lru_cache(maxsize=None)
def _make_sc_pool(vocab, emb, batch, seqlen, nc, ns):
    nw = nc * ns
    assert batch % nw == 0 and seqlen % 2 == 0 and emb % _LANES == 0
    s_per = batch // nw
    half = seqlen // 2
    rows_per = s_per * 2
    nvec = emb // _LANES
    unroll = 10
    assert half % unroll == 0
    mesh = plsc.VectorSubcoreMesh(core_axis_name="c", subcore_axis_name="s")

    @functools.partial(
        pl.kernel,
        out_type=jax.ShapeDtypeStruct((batch, emb), jnp.float32),
        mesh=mesh,
        scratch_types=[
            pltpu.VMEM((rows_per, half), jnp.int32),
            pltpu.VMEM((2, half, emb), jnp.float32),
            pltpu.VMEM((s_per, emb), jnp.float32),
            pltpu.SemaphoreType.DMA,
            pltpu.SemaphoreType.DMA,
        ],
        compiler_params=pltpu.CompilerParams(use_tc_tiling_on_sc=False),
    )
    def sc_pool(idx_hbm, table_hbm, sums_hbm, idx_v, rows_v, sums_v, sem0, sem1):
        wid = lax.axis_index("s") * nc + lax.axis_index("c")
        row_base = wid * rows_per
        pltpu.sync_copy(idx_hbm.at[pl.ds(row_base, rows_per)], idx_v)
        rows0 = rows_v.at[0]
        rows1 = rows_v.at[1]

        def fire(j, dst, sem):
            pltpu.async_copy(table_hbm.at[idx_v.at[j]], dst, sem)

        def wait(j, dst, sem):
            pltpu.make_async_copy(table_hbm.at[idx_v.at[j]], dst, sem).wait()

        def accum(rows, acc):
            def body(i, carry):
                carry = list(carry)
                r = i * unroll
                for k in range(unroll):
                    off = (k % 2) * nvec
                    for v in range(nvec):
                        carry[off + v] += rows[r + k, pl.ds(v * _LANES, _LANES)]
                return tuple(carry)

            return lax.fori_loop(0, half // unroll, body, acc)

        zero = jnp.zeros((_LANES,), jnp.float32)
        fire(0, rows0, sem0)

        def sentence(s, carry):
            j0 = 2 * s
            fire(j0 + 1, rows1, sem1)
            wait(j0, rows0, sem0)
            acc = accum(rows0, (zero,) * (2 * nvec))

            @pl.when(s < s_per - 1)
            def _():
                fire(j0 + 2, rows0, sem0)

            wait(j0 + 1, rows1, sem1)
            acc = accum(rows1, acc)
            for v in range(nvec):
                sums_v[s, pl.ds(v * _LANES, _LANES)] = acc[v] + acc[nvec + v]
            return carry

        lax.fori_loop(0, s_per, sentence, 0)
        pltpu.sync_copy(sums_v, sums_hbm.at[pl.ds(wid * s_per, s_per)])

    return sc_pool


@functools.lru_cache(maxsize=None)
def _make_tc_head(batch, seqlen, emb, nclass, pad):
    bb = 512
    assert batch % bb == 0

    def body(sent_ref, sums_ref, w_ref, b_ref, out_ref):
        cnt = jnp.sum((sent_ref[...] != pad).astype(jnp.float32), axis=1,
                      keepdims=True)
        pooled = sums_ref[...] / cnt
        logits = lax.dot_general(pooled, w_ref[...], (((1,), (1,)), ((), ())),
                                 preferred_element_type=jnp.float32)
        logits = logits + b_ref[...]
        shifted = logits - jnp.max(logits, axis=1, keepdims=True)
        lse = jnp.log(jnp.sum(jnp.exp(shifted), axis=1, keepdims=True))
        out_ref[...] = shifted - lse

    return pl.pallas_call(
        body,
        grid=(batch // bb,),
        in_specs=[
            pl.BlockSpec((bb, seqlen), lambda i: (i, 0)),
            pl.BlockSpec((bb, emb), lambda i: (i, 0)),
            pl.BlockSpec((nclass, emb), lambda i: (0, 0)),
            pl.BlockSpec((1, nclass), lambda i: (0, 0)),
        ],
        out_specs=pl.BlockSpec((bb, nclass), lambda i: (i, 0)),
        out_shape=jax.ShapeDtypeStruct((batch, nclass), jnp.float32),
    )


def kernel(sentences, emb_table, fc_w, fc_b):
    batch, seqlen = sentences.shape
    vocab, emb = emb_table.shape
    nclass = fc_w.shape[0]
    info = plsc.get_sparse_core_info()
    nc, ns = info.num_cores, info.num_subcores
    sent_i32 = sentences.astype(jnp.int32)
    idx2 = sent_i32.reshape(batch * 2, seqlen // 2)
    sums = _make_sc_pool(vocab, emb, batch, seqlen, nc, ns)(idx2, emb_table)
    head = _make_tc_head(batch, seqlen, emb, nclass, 0)
    return head(sent_i32, sums, fc_w, fc_b.reshape(1, nclass))
